# baseline (speedup 1.0000x reference)
import jax
import jax.numpy as jnp
from jax import lax
from jax.experimental import pallas as pl
from jax.experimental.pallas import tpu as pltpu
from jax.experimental.pallas import tpu_sc as plsc

VOCAB = 100001
EMBED_DIM = 32
BATCH = 16384
H1 = 64
H2 = 32

_INFO = plsc.get_sparse_core_info()
_NC, _NS = _INFO.num_cores, _INFO.num_subcores
_NW = _NC * _NS
_CHUNK = 4096


def _gatherT_body(
    tableT_hbm, idx_hbm, outT_hbm, row_v, idx_v, out0_v, out1_v, sem, osem
):
    c = lax.axis_index("s") * _NC + lax.axis_index("c")
    pltpu.async_copy(tableT_hbm.at[c], row_v, sem)
    pltpu.sync_copy(idx_hbm, idx_v)
    pltpu.make_async_copy(tableT_hbm.at[c], row_v, sem).wait()

    def step(t, _):
        for half, out_v in ((0, out0_v), (1, out1_v)):
            base = (2 * t + half) * _CHUNK

            @plsc.parallel_loop(0, _CHUNK // 16, unroll=8)
            def gather_grp(g):
                ids = idx_v[pl.ds(base + g * 16, 16)]
                vals = plsc.load_gather(row_v, [ids])
                out_v[pl.ds(g * 16, 16)] = vals

            @pl.when(t >= 1)
            def _drain():
                pltpu.make_async_copy(
                    out_v, outT_hbm.at[c, pl.ds(base, _CHUNK)], osem
                ).wait()

            pltpu.async_copy(out_v, outT_hbm.at[c, pl.ds(base, _CHUNK)], osem)
        return _

    lax.fori_loop(0, BATCH // (2 * _CHUNK), step, 0)
    pltpu.make_async_copy(out0_v, outT_hbm.at[c, pl.ds(0, _CHUNK)], osem).wait()
    pltpu.make_async_copy(out1_v, outT_hbm.at[c, pl.ds(0, _CHUNK)], osem).wait()


_sc_gatherT = pl.kernel(
    _gatherT_body,
    out_type=jax.ShapeDtypeStruct((EMBED_DIM, BATCH), jnp.float32),
    mesh=plsc.VectorSubcoreMesh(core_axis_name="c", subcore_axis_name="s"),
    scratch_types=[
        pltpu.VMEM((VOCAB,), jnp.float32),
        pltpu.VMEM((BATCH,), jnp.int32),
        pltpu.VMEM((_CHUNK,), jnp.float32),
        pltpu.VMEM((_CHUNK,), jnp.float32),
        pltpu.SemaphoreType.DMA,
        pltpu.SemaphoreType.DMA,
    ],
    compiler_params=pltpu.CompilerParams(needs_layout_passes=False),
)


_MLP_BLOCK = 16384


def _mlpT_body(x_ref, w1t_ref, b1_ref, w2t_ref, b2_ref, o_ref):
    h = jnp.maximum(
        jax.lax.dot_general(w1t_ref[...], x_ref[...], (((1,), (0,)), ((), ())),
                            preferred_element_type=jnp.float32)
        + b1_ref[...][:, None],
        0.0,
    )
    o_ref[...] = (
        jax.lax.dot_general(w2t_ref[...], h, (((1,), (0,)), ((), ())),
                            preferred_element_type=jnp.float32)
        + b2_ref[...][:, None]
    )


def _tc_mlpT(xT, w1t, b1, w2t, b2):
    grid = (BATCH // _MLP_BLOCK,)
    return pl.pallas_call(
        _mlpT_body,
        grid=grid,
        in_specs=[
            pl.BlockSpec((EMBED_DIM, _MLP_BLOCK), lambda i: (0, i)),
            pl.BlockSpec((H1, EMBED_DIM), lambda i: (0, 0)),
            pl.BlockSpec((H1,), lambda i: (0,)),
            pl.BlockSpec((H2, H1), lambda i: (0, 0)),
            pl.BlockSpec((H2,), lambda i: (0,)),
        ],
        out_specs=pl.BlockSpec((H2, _MLP_BLOCK), lambda i: (0, i)),
        out_shape=jax.ShapeDtypeStruct((H2, BATCH), jnp.float32),
    )(xT, w1t, b1, w2t, b2)


@jax.jit
def kernel(user_id, emb_table, W1, b1, W2, b2):
    gatheredT = _sc_gatherT(emb_table.T, user_id)
    outT = _tc_mlpT(gatheredT, W1.T, b1, W2.T, b2)
    return outT.T


# speedup vs baseline: 1.0054x; 1.0054x over previous
import jax
import jax.numpy as jnp
from jax import lax
from jax.experimental import pallas as pl
from jax.experimental.pallas import tpu as pltpu
from jax.experimental.pallas import tpu_sc as plsc

VOCAB = 100001
EMBED_DIM = 32
BATCH = 16384
H1 = 64
H2 = 32

_INFO = plsc.get_sparse_core_info()
_NC, _NS = _INFO.num_cores, _INFO.num_subcores
_NW = _NC * _NS
_CHUNK = 2048


def _gatherT_body(
    tableT_hbm, idx_hbm, outT_hbm, row_v, idx_v, out0_v, out1_v, sem, osem
):
    c = lax.axis_index("s") * _NC + lax.axis_index("c")
    pltpu.async_copy(tableT_hbm.at[c], row_v, sem)
    pltpu.sync_copy(idx_hbm, idx_v)
    pltpu.make_async_copy(tableT_hbm.at[c], row_v, sem).wait()

    def step(t, _):
        for half, out_v in ((0, out0_v), (1, out1_v)):
            base = (2 * t + half) * _CHUNK

            @plsc.parallel_loop(0, _CHUNK // 16, unroll=16)
            def gather_grp(g):
                ids = idx_v[pl.ds(base + g * 16, 16)]
                vals = plsc.load_gather(row_v, [ids])
                out_v[pl.ds(g * 16, 16)] = vals

            @pl.when(t >= 1)
            def _drain():
                pltpu.make_async_copy(
                    out_v, outT_hbm.at[c, pl.ds(base, _CHUNK)], osem
                ).wait()

            pltpu.async_copy(out_v, outT_hbm.at[c, pl.ds(base, _CHUNK)], osem)
        return _

    lax.fori_loop(0, BATCH // (2 * _CHUNK), step, 0)
    pltpu.make_async_copy(out0_v, outT_hbm.at[c, pl.ds(0, _CHUNK)], osem).wait()
    pltpu.make_async_copy(out1_v, outT_hbm.at[c, pl.ds(0, _CHUNK)], osem).wait()


_sc_gatherT = pl.kernel(
    _gatherT_body,
    out_type=jax.ShapeDtypeStruct((EMBED_DIM, BATCH), jnp.float32),
    mesh=plsc.VectorSubcoreMesh(core_axis_name="c", subcore_axis_name="s"),
    scratch_types=[
        pltpu.VMEM((VOCAB,), jnp.float32),
        pltpu.VMEM((BATCH,), jnp.int32),
        pltpu.VMEM((_CHUNK,), jnp.float32),
        pltpu.VMEM((_CHUNK,), jnp.float32),
        pltpu.SemaphoreType.DMA,
        pltpu.SemaphoreType.DMA,
    ],
    compiler_params=pltpu.CompilerParams(needs_layout_passes=False),
)


_MLP_BLOCK = 8192


def _mlpT_body(x_ref, w1t_ref, b1_ref, w2t_ref, b2_ref, o_ref):
    h = jnp.maximum(
        jax.lax.dot_general(w1t_ref[...], x_ref[...], (((1,), (0,)), ((), ())),
                            preferred_element_type=jnp.float32)
        + b1_ref[...][:, None],
        0.0,
    )
    o_ref[...] = (
        jax.lax.dot_general(w2t_ref[...], h, (((1,), (0,)), ((), ())),
                            preferred_element_type=jnp.float32)
        + b2_ref[...][:, None]
    )


def _tc_mlpT(xT, w1t, b1, w2t, b2):
    grid = (BATCH // _MLP_BLOCK,)
    return pl.pallas_call(
        _mlpT_body,
        grid=grid,
        in_specs=[
            pl.BlockSpec((EMBED_DIM, _MLP_BLOCK), lambda i: (0, i)),
            pl.BlockSpec((H1, EMBED_DIM), lambda i: (0, 0)),
            pl.BlockSpec((H1,), lambda i: (0,)),
            pl.BlockSpec((H2, H1), lambda i: (0, 0)),
            pl.BlockSpec((H2,), lambda i: (0,)),
        ],
        out_specs=pl.BlockSpec((H2, _MLP_BLOCK), lambda i: (0, i)),
        out_shape=jax.ShapeDtypeStruct((H2, BATCH), jnp.float32),
    )(xT, w1t, b1, w2t, b2)


@jax.jit
def kernel(user_id, emb_table, W1, b1, W2, b2):
    gatheredT = _sc_gatherT(emb_table.T, user_id)
    outT = _tc_mlpT(gatheredT, W1.T, b1, W2.T, b2)
    return outT.T
